# baseline (speedup 1.0000x reference)
import functools

import jax
import jax.numpy as jnp
from jax import lax
from jax.experimental import pallas as pl
from jax.experimental.pallas import tpu as pltpu
from jax.experimental.pallas import tpu_sc as plsc

N = 10000
D = 256
H = 128
E = 160000
NC = 2
NS = 16
CH = 80
EPT = E // NS
NCHUNK = EPT // CH
NP = 10240
RPT = NP // NS

NW = NC * NS
EPW = E // NW
NV = EPW // 16
REM = EPW - NV * 16


def _sc_agg(xr, colm, rowm, zrow):
    mesh = plsc.VectorSubcoreMesh(core_axis_name="c", subcore_axis_name="s")

    @functools.partial(
        pl.kernel,
        out_type=jax.ShapeDtypeStruct((NC, NP, H), jnp.float32),
        mesh=mesh,
        scratch_types=[
            pltpu.VMEM((4, CH), jnp.int32),
            pltpu.VMEM((4, CH), jnp.int32),
            pltpu.VMEM((CH, H), jnp.float32),
            pltpu.VMEM((CH, H), jnp.float32),
            pltpu.VMEM_SHARED((NP, H), jnp.float32),
            [pltpu.SemaphoreType.DMA] * 4,
            [pltpu.SemaphoreType.DMA] * 4,
            [pltpu.SemaphoreType.DMA] * 2,
            [pltpu.SemaphoreType.DMA] * 2,
        ],
    )
    def body(xr_hbm, colm_hbm, rowm_hbm, zrow_hbm,
             sums_hbm,
             gbuf, rbuf, buf0, buf1, acc,
             gisem, rsem, gsem, ssem):
        c = lax.axis_index("c")
        s = lax.axis_index("s")

        pltpu.sync_copy(zrow_hbm.at[pl.ds(s * RPT, RPT)],
                        acc.at[pl.ds(s * RPT, RPT)])

        def _cidx(j):
            return colm_hbm.at[pl.ds(s * EPT + j * CH, CH)]

        def _ridx(j):
            return rowm_hbm.at[pl.ds(s * EPT + j * CH, CH)]

        bufs = (buf0, buf1)

        def stage_idx(m, islot):
            pltpu.async_copy(_cidx(m), gbuf.at[islot], gisem[islot])
            pltpu.async_copy(_ridx(m), rbuf.at[islot], rsem[islot])

        def launch_gather(m, islot, b):
            pltpu.make_async_copy(_cidx(m), gbuf.at[islot], gisem[islot]).wait()

            @pl.loop(0, CH // 16)
            def _(q):
                v = gbuf[islot, pl.ds(q * 16, 16)]
                gbuf[islot, pl.ds(q * 16, 16)] = v * 2 + c

            pltpu.async_copy(xr_hbm.at[gbuf.at[islot]], bufs[b], gsem[b])

        def drain(m, islot, b):
            pltpu.make_async_copy(
                xr_hbm.at[gbuf.at[islot]], bufs[b], gsem[b]).wait()
            pltpu.make_async_copy(_ridx(m), rbuf.at[islot], rsem[islot]).wait()
            pltpu.async_copy(bufs[b], acc.at[rbuf.at[islot]], ssem[b], add=True)

        def wait_scatter(islot, b):
            pltpu.make_async_copy(bufs[b], acc.at[rbuf.at[islot]], ssem[b]).wait()

        for k in range(4):
            stage_idx(k, k)
        for k in range(2):
            launch_gather(k, k, k)

        @pl.loop(0, NCHUNK, step=4)
        def _(j):
            for half in range(2):
                for k in (2 * half, 2 * half + 1):
                    @pl.when(j + k < NCHUNK)
                    def _(k=k):
                        drain(j + k, k, k % 2)
                for k in (2 * half, 2 * half + 1):
                    @pl.when(j + k < NCHUNK)
                    def _(k=k):
                        wait_scatter(k, k % 2)

                        @pl.when(j + k + 4 < NCHUNK)
                        def _(k=k):
                            stage_idx(j + k + 4, k)

                        @pl.when(j + k + 2 < NCHUNK)
                        def _(k=k):
                            launch_gather(j + k + 2, (k + 2) % 4, k % 2)

        plsc.subcore_barrier()

        pltpu.sync_copy(acc.at[pl.ds(s * RPT, RPT)],
                        sums_hbm.at[c, pl.ds(s * RPT, RPT)])

    return body(xr, colm, rowm, zrow)


def _sc_count(rowf):
    mesh = plsc.VectorSubcoreMesh(core_axis_name="c", subcore_axis_name="s")

    @functools.partial(
        pl.kernel,
        out_type=jax.ShapeDtypeStruct((NW, NP), jnp.float32),
        mesh=mesh,
        scratch_types=[
            pltpu.VMEM((EPW,), jnp.int32),
            pltpu.VMEM((NP,), jnp.float32),
        ],
        compiler_params=pltpu.CompilerParams(needs_layout_passes=False),
    )
    def body(row_hbm, cnt_hbm, ridx, cl):
        c = lax.axis_index("c")
        s = lax.axis_index("s")
        w = c * NS + s

        zero16 = jnp.zeros((16,), jnp.float32)

        @pl.loop(0, NP // 16)
        def _(i):
            cl[pl.ds(i * 16, 16)] = zero16

        pltpu.sync_copy(row_hbm.at[pl.ds(w * EPW, EPW)], ridx)

        one16 = jnp.ones((16,), jnp.float32)

        @pl.loop(0, NV)
        def _(k):
            idx = ridx[pl.ds(k * 16, 16)]
            plsc.addupdate_scatter(cl, [idx], one16)

        if REM:
            lane = lax.iota(jnp.int32, 16)
            idx = ridx[pl.ds(EPW - 16, 16)]
            plsc.addupdate_scatter(cl, [idx], one16, mask=lane >= 16 - REM)

        pltpu.sync_copy(cl, cnt_hbm.at[w])

    return body(rowf)


BM = 1024


def _tc_update(sums2, cnt2, x, w, b2):
    def body(s_ref, c_ref, x_ref, w_ref, b_ref, o_ref):
        s0 = s_ref[0]
        s1 = s_ref[1]
        xb = x_ref[...]
        cnt = jnp.sum(c_ref[...], axis=0)[:, None] + 1.0
        inv = 1.0 / cnt
        r0 = (s0 + 2.0 * xb[:, :H]) * inv
        r1 = (s1 + 2.0 * xb[:, H:]) * inv
        acc = jnp.dot(r0, w_ref[0:H, :], preferred_element_type=jnp.float32)
        acc += jnp.dot(r1, w_ref[H:2 * H, :], preferred_element_type=jnp.float32)
        acc += jnp.dot(xb, w_ref[2 * H:, :], preferred_element_type=jnp.float32)
        o_ref[...] = jnp.maximum(acc + b_ref[...], 0.0)

    return pl.pallas_call(
        body,
        grid=(pl.cdiv(N, BM),),
        in_specs=[
            pl.BlockSpec((NC, BM, H), lambda i: (0, i, 0)),
            pl.BlockSpec((NW, BM), lambda i: (0, i)),
            pl.BlockSpec((BM, D), lambda i: (i, 0)),
            pl.BlockSpec((2 * D, D), lambda i: (0, 0)),
            pl.BlockSpec((1, D), lambda i: (0, 0)),
        ],
        out_specs=pl.BlockSpec((BM, D), lambda i: (i, 0)),
        out_shape=jax.ShapeDtypeStruct((N, D), jnp.float32),
    )(sums2, cnt2, x, w, b2)


def kernel(x, edge_index, edge_weight, kernel, bias):
    del edge_weight
    col = edge_index[1].astype(jnp.int32)
    row = edge_index[0].astype(jnp.int32)
    rowf = edge_index[0].astype(jnp.int32)
    xr = x.reshape(2 * N, H)
    zrow = jnp.zeros((NP, H), jnp.float32)
    sums2 = _sc_agg(xr, col, row, zrow)
    cnt2 = _sc_count(rowf)
    return _tc_update(sums2, cnt2, x, kernel, bias.reshape(1, D))


# speedup vs baseline: 1.0496x; 1.0496x over previous
import functools

import jax
import jax.numpy as jnp
from jax import lax
from jax.experimental import pallas as pl
from jax.experimental.pallas import tpu as pltpu
from jax.experimental.pallas import tpu_sc as plsc

N = 10000
D = 256
H = 128
E = 160000
NC = 2
NS = 16
CH = 40
EPT = E // NS
NCHUNK = EPT // CH
NP = 10240
RPT = NP // NS

NW = NC * NS
EPW = E // NW
NV = EPW // 16
REM = EPW - NV * 16


def _sc_agg(xr, colm, rowm, zrow):
    mesh = plsc.VectorSubcoreMesh(core_axis_name="c", subcore_axis_name="s")

    @functools.partial(
        pl.kernel,
        out_type=jax.ShapeDtypeStruct((NC, NP, H), jnp.float32),
        mesh=mesh,
        scratch_types=[
            pltpu.VMEM((EPT,), jnp.int32),
            pltpu.VMEM((3, CH), jnp.int32),
            pltpu.VMEM((CH, H), jnp.float32),
            pltpu.VMEM((CH, H), jnp.float32),
            pltpu.VMEM((CH, H), jnp.float32),
            pltpu.VMEM_SHARED((NP, H), jnp.float32),
            pltpu.SemaphoreType.DMA,
            pltpu.SemaphoreType.DMA,
            pltpu.SemaphoreType.DMA,
            pltpu.SemaphoreType.DMA,
            pltpu.SemaphoreType.DMA,
            pltpu.SemaphoreType.DMA,
            pltpu.SemaphoreType.DMA,
            pltpu.SemaphoreType.DMA,
            pltpu.SemaphoreType.DMA,
        ],
    )
    def body(xr_hbm, colm_hbm, rowm_hbm, zrow_hbm,
             sums_hbm,
             gidx, rbuf, buf0, buf1, buf2, acc,
             g0, g1, g2, r0, r1, r2, s0, s1, s2):
        c = lax.axis_index("c")
        s = lax.axis_index("s")

        pltpu.sync_copy(zrow_hbm.at[pl.ds(s * RPT, RPT)],
                        acc.at[pl.ds(s * RPT, RPT)])

        pltpu.sync_copy(colm_hbm.at[pl.ds(s * EPT, EPT)], gidx)

        @pl.loop(0, EPT // 16)
        def _(k):
            v = gidx[pl.ds(k * 16, 16)]
            gidx[pl.ds(k * 16, 16)] = v * 2 + c

        plsc.subcore_barrier()

        def _gidx(j):
            return gidx.at[pl.ds(j * CH, CH)]

        def _ridx(j):
            return rowm_hbm.at[pl.ds(s * EPT + j * CH, CH)]

        bufs = (buf0, buf1, buf2)
        gsem = (g0, g1, g2)
        rsem = (r0, r1, r2)
        ssem = (s0, s1, s2)

        for k in range(3):
            pltpu.async_copy(xr_hbm.at[_gidx(k)], bufs[k], gsem[k])
            pltpu.async_copy(_ridx(k), rbuf.at[k], rsem[k])

        @pl.loop(0, NCHUNK, step=3)
        def _(j):
            for k in range(3):
                @pl.when(j + k < NCHUNK)
                def _(k=k):
                    pltpu.make_async_copy(
                        xr_hbm.at[_gidx(j + k)], bufs[k], gsem[k]).wait()
                    pltpu.make_async_copy(
                        _ridx(j + k), rbuf.at[k], rsem[k]).wait()
                    pltpu.async_copy(
                        bufs[k], acc.at[rbuf.at[k]], ssem[k], add=True)

            for k in range(3):
                @pl.when(j + k < NCHUNK)
                def _(k=k):
                    pltpu.make_async_copy(
                        bufs[k], acc.at[rbuf.at[k]], ssem[k]).wait()

                    @pl.when(j + k + 3 < NCHUNK)
                    def _(k=k):
                        pltpu.async_copy(
                            xr_hbm.at[_gidx(j + k + 3)], bufs[k], gsem[k])
                        pltpu.async_copy(_ridx(j + k + 3), rbuf.at[k], rsem[k])

        plsc.subcore_barrier()

        pltpu.sync_copy(acc.at[pl.ds(s * RPT, RPT)],
                        sums_hbm.at[c, pl.ds(s * RPT, RPT)])

    return body(xr, colm, rowm, zrow)


def _sc_count(rowf):
    mesh = plsc.VectorSubcoreMesh(core_axis_name="c", subcore_axis_name="s")

    @functools.partial(
        pl.kernel,
        out_type=jax.ShapeDtypeStruct((NW, NP), jnp.float32),
        mesh=mesh,
        scratch_types=[
            pltpu.VMEM((EPW,), jnp.int32),
            pltpu.VMEM((NP,), jnp.float32),
        ],
        compiler_params=pltpu.CompilerParams(needs_layout_passes=False),
    )
    def body(row_hbm, cnt_hbm, ridx, cl):
        c = lax.axis_index("c")
        s = lax.axis_index("s")
        w = c * NS + s

        zero16 = jnp.zeros((16,), jnp.float32)

        @pl.loop(0, NP // 16)
        def _(i):
            cl[pl.ds(i * 16, 16)] = zero16

        pltpu.sync_copy(row_hbm.at[pl.ds(w * EPW, EPW)], ridx)

        one16 = jnp.ones((16,), jnp.float32)

        @pl.loop(0, NV)
        def _(k):
            idx = ridx[pl.ds(k * 16, 16)]
            plsc.addupdate_scatter(cl, [idx], one16)

        if REM:
            lane = lax.iota(jnp.int32, 16)
            idx = ridx[pl.ds(EPW - 16, 16)]
            plsc.addupdate_scatter(cl, [idx], one16, mask=lane >= 16 - REM)

        pltpu.sync_copy(cl, cnt_hbm.at[w])

    return body(rowf)


BM = 1024


def _tc_update(sums2, cnt2, x, w, b2):
    def body(s_ref, c_ref, x_ref, w_ref, b_ref, o_ref):
        s0 = s_ref[0]
        s1 = s_ref[1]
        xb = x_ref[...]
        cnt = jnp.sum(c_ref[...], axis=0)[:, None] + 1.0
        inv = 1.0 / cnt
        r0 = (s0 + 2.0 * xb[:, :H]) * inv
        r1 = (s1 + 2.0 * xb[:, H:]) * inv
        acc = jnp.dot(r0, w_ref[0:H, :], preferred_element_type=jnp.float32)
        acc += jnp.dot(r1, w_ref[H:2 * H, :], preferred_element_type=jnp.float32)
        acc += jnp.dot(xb, w_ref[2 * H:, :], preferred_element_type=jnp.float32)
        o_ref[...] = jnp.maximum(acc + b_ref[...], 0.0)

    return pl.pallas_call(
        body,
        grid=(pl.cdiv(N, BM),),
        in_specs=[
            pl.BlockSpec((NC, BM, H), lambda i: (0, i, 0)),
            pl.BlockSpec((NW, BM), lambda i: (0, i)),
            pl.BlockSpec((BM, D), lambda i: (i, 0)),
            pl.BlockSpec((2 * D, D), lambda i: (0, 0)),
            pl.BlockSpec((1, D), lambda i: (0, 0)),
        ],
        out_specs=pl.BlockSpec((BM, D), lambda i: (i, 0)),
        out_shape=jax.ShapeDtypeStruct((N, D), jnp.float32),
    )(sums2, cnt2, x, w, b2)


def kernel(x, edge_index, edge_weight, kernel, bias):
    del edge_weight
    col = edge_index[1].astype(jnp.int32)
    row = edge_index[0].astype(jnp.int32)
    rowf = edge_index[0].astype(jnp.int32)
    xr = x.reshape(2 * N, H)
    zrow = jnp.zeros((NP, H), jnp.float32)
    sums2 = _sc_agg(xr, col, row, zrow)
    cnt2 = _sc_count(rowf)
    return _tc_update(sums2, cnt2, x, kernel, bias.reshape(1, D))
